# baseline (speedup 1.0000x reference)
import jax
import jax.numpy as jnp
from jax.experimental import pallas as pl
from jax.experimental.pallas import tpu as pltpu

_L, _K, _D = 32, 1024, 2048
_LAMBDA = 0.1
_CC_LOW = 768
_CC_HIGH = 1024
_TOTAL_UNITS = 16 * _CC_LOW + 16 * _CC_HIGH


def _body(x, kk):
    row_sum = jnp.sum(x, axis=1)
    col = jax.lax.broadcasted_iota(jnp.int32, x.shape, 1)
    r_dk = jnp.sum(jnp.where(col == kk[:, None], x, 0.0), axis=1)
    r_minus = (row_sum - r_dk) * jnp.float32(1.0 / (_D - 1))
    a = jnp.abs(r_dk)
    b = jnp.abs(r_minus)
    return jnp.sum((a - b) / (a + b + jnp.float32(1e-6)))


def _low_kernel(x_ref, out_ref):
    @pl.when(pl.program_id(0) == 0)
    def _init():
        out_ref[0, 0] = jnp.float32(0.0)

    r = jax.lax.broadcasted_iota(jnp.int32, (4 * _CC_LOW,), 0)
    kk = jnp.remainder(r, _CC_LOW)
    x = x_ref[...].reshape(4 * _CC_LOW, _D)
    out_ref[0, 0] += _body(x, kk)


def _high_kernel(part_ref, x_ref, out_ref):
    @pl.when(pl.program_id(0) == 0)
    def _init():
        out_ref[0, 0] = part_ref[0, 0]

    kk = jnp.bitwise_and(
        jax.lax.broadcasted_iota(jnp.int32, (2 * _CC_HIGH,), 0), _K - 1
    )
    x = x_ref[...].reshape(2 * _CC_HIGH, _D)
    out_ref[0, 0] += _body(x, kk)


def kernel(R):
    part = pl.pallas_call(
        _low_kernel,
        grid=(4,),
        in_specs=[pl.BlockSpec((4, _CC_LOW, _D), lambda l: (l, 0, 0))],
        out_specs=pl.BlockSpec((1, 1), lambda l: (0, 0),
                               memory_space=pltpu.SMEM),
        out_shape=jax.ShapeDtypeStruct((1, 1), jnp.float32),
    )(R)
    total = pl.pallas_call(
        _high_kernel,
        grid=(8,),
        in_specs=[
            pl.BlockSpec(memory_space=pltpu.SMEM),
            pl.BlockSpec((2, _CC_HIGH, _D), lambda l: (l + 8, 0, 0)),
        ],
        out_specs=pl.BlockSpec((1, 1), lambda l: (0, 0),
                               memory_space=pltpu.SMEM),
        out_shape=jax.ShapeDtypeStruct((1, 1), jnp.float32),
    )(part, R)
    return total[0, 0] * jnp.float32(-_LAMBDA / _TOTAL_UNITS)


# speedup vs baseline: 1.0721x; 1.0721x over previous
import jax
import jax.numpy as jnp
from jax.experimental import pallas as pl
from jax.experimental.pallas import tpu as pltpu

_L, _K, _D = 32, 1024, 2048
_LAMBDA = 0.1
_CC_LOW = 768
_CC_HIGH = 1024
_TOTAL_UNITS = 16 * _CC_LOW + 16 * _CC_HIGH


def _body(x, kk):
    row_sum = jnp.sum(x, axis=1)
    col = jax.lax.broadcasted_iota(jnp.int32, x.shape, 1)
    r_dk = jnp.sum(jnp.where(col == kk[:, None], x, 0.0), axis=1)
    r_minus = (row_sum - r_dk) * jnp.float32(1.0 / (_D - 1))
    a = jnp.abs(r_dk)
    b = jnp.abs(r_minus)
    return jnp.sum((a - b) / (a + b + jnp.float32(1e-6)))


def _fused_kernel(lo_ref, hi_ref, out_ref):
    @pl.when(pl.program_id(0) == 0)
    def _init():
        out_ref[0, 0] = jnp.float32(0.0)

    r = jax.lax.broadcasted_iota(jnp.int32, (2 * _CC_LOW,), 0)
    kk_lo = jnp.where(r >= _CC_LOW, r - _CC_LOW, r)
    lo = lo_ref[...].reshape(2 * _CC_LOW, _D)
    kk_hi = jnp.bitwise_and(
        jax.lax.broadcasted_iota(jnp.int32, (2 * _CC_HIGH,), 0), _K - 1
    )
    hi = hi_ref[...].reshape(2 * _CC_HIGH, _D)
    out_ref[0, 0] += _body(lo, kk_lo) + _body(hi, kk_hi)


def kernel(R):
    total = pl.pallas_call(
        _fused_kernel,
        grid=(8,),
        in_specs=[
            pl.BlockSpec((2, _CC_LOW, _D), lambda l: (l, 0, 0)),
            pl.BlockSpec((2, _CC_HIGH, _D), lambda l: (l + 8, 0, 0)),
        ],
        out_specs=pl.BlockSpec((1, 1), lambda l: (0, 0),
                               memory_space=pltpu.SMEM),
        out_shape=jax.ShapeDtypeStruct((1, 1), jnp.float32),
    )(R, R)
    return total[0, 0] * jnp.float32(-_LAMBDA / _TOTAL_UNITS)


# speedup vs baseline: 1.1090x; 1.0344x over previous
import jax
import jax.numpy as jnp
from jax.experimental import pallas as pl
from jax.experimental.pallas import tpu as pltpu

_L, _K, _D = 32, 1024, 2048
_LAMBDA = 0.1
_CC_LOW = 768
_CC_HIGH = 1024
_TOTAL_UNITS = 16 * _CC_LOW + 16 * _CC_HIGH


def _body(x, kk):
    row_sum = jnp.sum(x, axis=1)
    col = jax.lax.broadcasted_iota(jnp.int32, x.shape, 1)
    r_dk = jnp.sum(jnp.where(col == kk[:, None], x, 0.0), axis=1)
    r_minus = (row_sum - r_dk) * jnp.float32(1.0 / (_D - 1))
    a = jnp.abs(r_dk)
    b = jnp.abs(r_minus)
    return jnp.sum((a - b) / (a + b + jnp.float32(1e-6)))


def _fused_kernel(lo_ref, hi_ref, out_ref):
    @pl.when(pl.program_id(0) == 0)
    def _init():
        out_ref[0, 0] = jnp.float32(0.0)

    kk_lo = jax.lax.broadcasted_iota(jnp.int32, (_CC_LOW,), 0)
    lo = lo_ref[...].reshape(_CC_LOW, _D)
    kk_hi = jax.lax.broadcasted_iota(jnp.int32, (_CC_HIGH,), 0)
    hi = hi_ref[...].reshape(_CC_HIGH, _D)
    out_ref[0, 0] += _body(lo, kk_lo) + _body(hi, kk_hi)


def kernel(R):
    total = pl.pallas_call(
        _fused_kernel,
        grid=(16,),
        in_specs=[
            pl.BlockSpec((1, _CC_LOW, _D), lambda l: (l, 0, 0)),
            pl.BlockSpec((1, _CC_HIGH, _D), lambda l: (l + 16, 0, 0)),
        ],
        out_specs=pl.BlockSpec((1, 1), lambda l: (0, 0),
                               memory_space=pltpu.SMEM),
        out_shape=jax.ShapeDtypeStruct((1, 1), jnp.float32),
    )(R, R)
    return total[0, 0] * jnp.float32(-_LAMBDA / _TOTAL_UNITS)
